# baseline (speedup 1.0000x reference)
import functools

import jax
import jax.numpy as jnp
from jax import lax
from jax.experimental import pallas as pl
from jax.experimental.pallas import tpu as pltpu
from jax.experimental.pallas import tpu_sc as plsc

N = 10000
E = 160000
D_NODE = 32
D_EDGE = 16
D_OUT = 32
H = D_OUT * D_NODE

NC = 2
NS = 16
NW = NC * NS
CB = 40
N_PAD = 10240
NPS = N_PAD // NS

E_SPLIT = ((40960, 128, 2560), (40960, 128, 2560), (40960, 128, 2560),
           (37120, 40, 1280))

_SC_MESH = dict(core_axis_name="c", subcore_axis_name="s")


def _sc_gather(node_pad, src, e0, cnt, cb, order_dep):
    nch = cnt // (NW * cb)
    epw = nch * cb

    @functools.partial(
        pl.kernel,
        mesh=plsc.VectorSubcoreMesh(**_SC_MESH),
        out_type=jax.ShapeDtypeStruct((cnt, 128), jnp.float32),
        scratch_types=[
            pltpu.VMEM((epw,), jnp.int32),
            pltpu.VMEM((2, cb, 128), jnp.float32),
            pltpu.SemaphoreType.DMA,
            pltpu.SemaphoreType.DMA,
        ],
    )
    def k(node_hbm, src_hbm, dep_hbm, out_hbm, idx_v, rows_v, sg0, sg1):
        del dep_hbm
        cid = lax.axis_index("c")
        sid = lax.axis_index("s")
        wid = sid * NC + cid
        base = wid * epw
        pltpu.sync_copy(src_hbm.at[pl.ds(e0 + base, epw)], idx_v)
        pltpu.async_copy(node_hbm.at[idx_v.at[pl.ds(0, cb)]], rows_v.at[0],
                         sg0)

        def pair(p, _):
            j0 = 2 * p
            j1 = j0 + 1
            pltpu.async_copy(node_hbm.at[idx_v.at[pl.ds(j1 * cb, cb)]],
                             rows_v.at[1], sg1)
            pltpu.make_async_copy(node_hbm.at[idx_v.at[pl.ds(j0 * cb, cb)]],
                                  rows_v.at[0], sg0).wait()
            pltpu.sync_copy(rows_v.at[0], out_hbm.at[pl.ds(base + j0 * cb, cb)])

            @pl.when(j1 + 1 < nch)
            def _():
                pltpu.async_copy(
                    node_hbm.at[idx_v.at[pl.ds((j1 + 1) * cb, cb)]],
                    rows_v.at[0], sg0)

            pltpu.make_async_copy(node_hbm.at[idx_v.at[pl.ds(j1 * cb, cb)]],
                                  rows_v.at[1], sg1).wait()
            pltpu.sync_copy(rows_v.at[1], out_hbm.at[pl.ds(base + j1 * cb, cb)])
            return 0

        lax.fori_loop(0, nch // 2, pair, 0)
        if nch % 2:
            j = nch - 1
            pltpu.make_async_copy(node_hbm.at[idx_v.at[pl.ds(j * cb, cb)]],
                                  rows_v.at[0], sg0).wait()
            pltpu.sync_copy(rows_v.at[0], out_hbm.at[pl.ds(base + j * cb, cb)])

    return k(node_pad, src, order_dep)


def _sc_scatter(msg, dst_h, cnt, cb, prev, order_dep):
    nch = cnt // (NW * cb)
    epw = nch * cb
    dst_r = dst_h.reshape(NW, nch, cb)

    @functools.partial(
        pl.kernel,
        mesh=plsc.VectorSubcoreMesh(**_SC_MESH),
        out_type=jax.ShapeDtypeStruct((NC, N_PAD, 128), jnp.float32),
        scratch_types=[
            pltpu.VMEM((nch, cb), jnp.int32),
            pltpu.VMEM((2, cb, 128), jnp.float32),
            pltpu.VMEM_SHARED((N_PAD, 128), jnp.float32),
            pltpu.SemaphoreType.DMA,
            pltpu.SemaphoreType.DMA,
        ],
    )
    def k(msg_hbm, dst_hbm, prev_hbm, dep_hbm, out_hbm, idx_v, msg_v, acc_sh,
          sl0, sl1):
        del dep_hbm
        cid = lax.axis_index("c")
        sid = lax.axis_index("s")
        wid = sid * NC + cid
        base = wid * epw
        pltpu.sync_copy(prev_hbm.at[cid, pl.ds(sid * NPS, NPS)],
                        acc_sh.at[pl.ds(sid * NPS, NPS)])
        plsc.subcore_barrier()
        pltpu.sync_copy(dst_hbm.at[wid], idx_v)
        pltpu.async_copy(msg_hbm.at[pl.ds(base, cb)], msg_v.at[0], sl0)

        def pair(p, _):
            j0 = 2 * p
            j1 = j0 + 1
            pltpu.async_copy(msg_hbm.at[pl.ds(base + j1 * cb, cb)],
                             msg_v.at[1], sl1)
            pltpu.make_async_copy(msg_hbm.at[pl.ds(base + j0 * cb, cb)],
                                  msg_v.at[0], sl0).wait()
            pltpu.sync_copy(msg_v.at[0], acc_sh.at[idx_v.at[j0]], add=True)

            @pl.when(j1 + 1 < nch)
            def _():
                pltpu.async_copy(msg_hbm.at[pl.ds(base + (j1 + 1) * cb, cb)],
                                 msg_v.at[0], sl0)

            pltpu.make_async_copy(msg_hbm.at[pl.ds(base + j1 * cb, cb)],
                                  msg_v.at[1], sl1).wait()
            pltpu.sync_copy(msg_v.at[1], acc_sh.at[idx_v.at[j1]], add=True)
            return 0

        lax.fori_loop(0, nch // 2, pair, 0)
        if nch % 2:
            j = nch - 1
            pltpu.make_async_copy(msg_hbm.at[pl.ds(base + j * cb, cb)],
                                  msg_v.at[0], sl0).wait()
            pltpu.sync_copy(msg_v.at[0], acc_sh.at[idx_v.at[j]], add=True)
        plsc.subcore_barrier()
        pltpu.sync_copy(acc_sh.at[pl.ds(sid * NPS, NPS)],
                        out_hbm.at[cid, pl.ds(sid * NPS, NPS)])

    return k(msg, dst_r, prev, order_dep)


def _msg_body(ef_ref, xs_ref, w1_ref, b1_ref, w2_ref, b2_ref, erep_ref,
              esum_ref, out_ref):
    h = jnp.maximum(
        jnp.dot(ef_ref[...], w1_ref[...], preferred_element_type=jnp.float32)
        + b1_ref[...], 0.0)
    we = jnp.dot(h.astype(jnp.bfloat16), w2_ref[...],
                 preferred_element_type=jnp.float32).astype(jnp.bfloat16) + b2_ref[...]
    xr = jnp.dot(xs_ref[...].astype(jnp.bfloat16), erep_ref[...],
                 preferred_element_type=jnp.float32).astype(jnp.bfloat16)
    out_ref[...] = jnp.dot(we * xr, esum_ref[...],
                           preferred_element_type=jnp.float32)


def _tc_msg(edge_feats, xs_h, e0, cnt, be, W1, b1, W2, b2):
    grid = (cnt // be,)
    f = jnp.arange(H)
    erep = (f[None, :] // D_OUT == jnp.arange(128)[:, None]).astype(jnp.bfloat16)
    esum = (f[:, None] % D_OUT == jnp.arange(128)[None, :]).astype(jnp.bfloat16)
    return pl.pallas_call(
        _msg_body,
        grid=grid,
        in_specs=[
            pl.BlockSpec((be, D_EDGE), lambda i, e0=e0, be=be: (i + e0 // be, 0)),
            pl.BlockSpec((be, 128), lambda i: (i, 0)),
            pl.BlockSpec((D_EDGE, H), lambda i: (0, 0)),
            pl.BlockSpec((1, H), lambda i: (0, 0)),
            pl.BlockSpec((H, H), lambda i: (0, 0)),
            pl.BlockSpec((1, H), lambda i: (0, 0)),
            pl.BlockSpec((128, H), lambda i: (0, 0)),
            pl.BlockSpec((H, 128), lambda i: (0, 0)),
        ],
        out_specs=pl.BlockSpec((be, 128), lambda i: (i, 0)),
        out_shape=jax.ShapeDtypeStruct((cnt, 128), jnp.float32),
    )(edge_feats, xs_h, W1.T, b1[None, :], W2.T.astype(jnp.bfloat16),
      b2[None, :].astype(jnp.bfloat16), erep, esum)


def _finish_body(aggp_ref, nf_ref, bconv_ref, wr_ref, wz_ref, wn_ref,
                 br_ref, bz_ref, bni_ref, bnh_ref, wres_ref, bres_ref,
                 gamma_ref, beta_ref, out_ref):
    agg = aggp_ref[0, :N, :D_OUT] + aggp_ref[1, :N, :D_OUT]
    x = jnp.maximum(agg + bconv_ref[...], 0.0)
    r = jax.nn.sigmoid(
        jnp.dot(x, wr_ref[...], preferred_element_type=jnp.float32) + br_ref[...])
    z = jax.nn.sigmoid(
        jnp.dot(x, wz_ref[...], preferred_element_type=jnp.float32) + bz_ref[...])
    n = jnp.tanh(
        jnp.dot(x, wn_ref[...], preferred_element_type=jnp.float32)
        + bni_ref[...] + r * bnh_ref[...])
    hnew = (1.0 - z) * n
    out = hnew + jnp.dot(nf_ref[...], wres_ref[...],
                         preferred_element_type=jnp.float32) + bres_ref[...]
    mean = jnp.mean(out, axis=0, keepdims=True)
    var = jnp.mean((out - mean) ** 2, axis=0, keepdims=True)
    out_ref[...] = (out - mean) * lax.rsqrt(var + 1e-5) * gamma_ref[...] + beta_ref[...]


def _tc_finish(aggp, node_feats, b_conv, Wih, Whh, bih, bhh, Wres,
               bres, gamma, beta):
    O = D_OUT
    args = (
        aggp, node_feats, b_conv[None, :],
        Wih[:O].T, Wih[O:2 * O].T, Wih[2 * O:].T,
        (bih[:O] + bhh[:O])[None, :],
        (bih[O:2 * O] + bhh[O:2 * O])[None, :],
        bih[2 * O:][None, :], bhh[2 * O:][None, :],
        Wres.T, bres[None, :], gamma[None, :], beta[None, :],
    )
    return pl.pallas_call(
        _finish_body,
        out_shape=jax.ShapeDtypeStruct((N, D_OUT), jnp.float32),
    )(*args)


def kernel(node_feats, edge_feats, edge_index, W1, b1, W2, b2, b_conv, Wih,
           Whh, bih, bhh, Wres, bres, gamma, beta):
    src = edge_index[0]
    dst = edge_index[1]
    node_pad = jnp.pad(node_feats, ((0, 0), (0, 128 - D_NODE)))

    offs = [0]
    for c, _, _ in E_SPLIT:
        offs.append(offs[-1] + c)
    xss, msgs = [], []
    for i, (cnt, cb, be) in enumerate(E_SPLIT):
        dep = xss[-1] if xss else src
        xss.append(_sc_gather(node_pad, src, offs[i], cnt, cb, dep))
    for i, (cnt, cb, be) in enumerate(E_SPLIT):
        msgs.append(_tc_msg(edge_feats, xss[i], offs[i], cnt, be, W1, b1,
                            W2, b2))
    aggp = jnp.zeros((NC, N_PAD, 128), dtype=jnp.float32)
    for i, (cnt, cb, be) in enumerate(E_SPLIT):
        aggp = _sc_scatter(msgs[i], dst[offs[i]:offs[i + 1]], cnt, cb, aggp,
                           xss[-1])
    return _tc_finish(aggp, node_feats, b_conv, Wih, Whh, bih, bhh,
                      Wres, bres, gamma, beta)


# speedup vs baseline: 1.0538x; 1.0538x over previous
import functools

import jax
import jax.numpy as jnp
from jax import lax
from jax.experimental import pallas as pl
from jax.experimental.pallas import tpu as pltpu
from jax.experimental.pallas import tpu_sc as plsc

N = 10000
E = 160000
D_NODE = 32
D_EDGE = 16
D_OUT = 32
H = D_OUT * D_NODE

NC = 2
NS = 16
NW = NC * NS
CB = 40
N_PAD = 10240
NPS = N_PAD // NS

E_SPLIT = ((40960, 128, 2560), (40960, 128, 2560), (40960, 128, 2560),
           (37120, 40, 1280))

_SC_MESH = dict(core_axis_name="c", subcore_axis_name="s")


def _sc_gather(node_pad, src, e0, cnt, cb):
    nch = cnt // (NW * cb)
    epw = nch * cb

    @functools.partial(
        pl.kernel,
        mesh=plsc.VectorSubcoreMesh(**_SC_MESH),
        out_type=jax.ShapeDtypeStruct((cnt, 128), jnp.float32),
        scratch_types=[
            pltpu.VMEM((epw,), jnp.int32),
            pltpu.VMEM((2, cb, 128), jnp.float32),
            pltpu.SemaphoreType.DMA,
            pltpu.SemaphoreType.DMA,
        ],
    )
    def k(node_hbm, src_hbm, out_hbm, idx_v, rows_v, sg0, sg1):
        cid = lax.axis_index("c")
        sid = lax.axis_index("s")
        wid = sid * NC + cid
        base = wid * epw
        pltpu.sync_copy(src_hbm.at[pl.ds(e0 + base, epw)], idx_v)
        pltpu.async_copy(node_hbm.at[idx_v.at[pl.ds(0, cb)]], rows_v.at[0],
                         sg0)

        def pair(p, _):
            j0 = 2 * p
            j1 = j0 + 1
            pltpu.async_copy(node_hbm.at[idx_v.at[pl.ds(j1 * cb, cb)]],
                             rows_v.at[1], sg1)
            pltpu.make_async_copy(node_hbm.at[idx_v.at[pl.ds(j0 * cb, cb)]],
                                  rows_v.at[0], sg0).wait()
            pltpu.sync_copy(rows_v.at[0], out_hbm.at[pl.ds(base + j0 * cb, cb)])

            @pl.when(j1 + 1 < nch)
            def _():
                pltpu.async_copy(
                    node_hbm.at[idx_v.at[pl.ds((j1 + 1) * cb, cb)]],
                    rows_v.at[0], sg0)

            pltpu.make_async_copy(node_hbm.at[idx_v.at[pl.ds(j1 * cb, cb)]],
                                  rows_v.at[1], sg1).wait()
            pltpu.sync_copy(rows_v.at[1], out_hbm.at[pl.ds(base + j1 * cb, cb)])
            return 0

        lax.fori_loop(0, nch // 2, pair, 0)
        if nch % 2:
            j = nch - 1
            pltpu.make_async_copy(node_hbm.at[idx_v.at[pl.ds(j * cb, cb)]],
                                  rows_v.at[0], sg0).wait()
            pltpu.sync_copy(rows_v.at[0], out_hbm.at[pl.ds(base + j * cb, cb)])

    return k(node_pad, src)


def _sc_scatter(msg, dst_h, cnt, cb, prev):
    nch = cnt // (NW * cb)
    epw = nch * cb
    dst_r = dst_h.reshape(NW, nch, cb)

    @functools.partial(
        pl.kernel,
        mesh=plsc.VectorSubcoreMesh(**_SC_MESH),
        out_type=jax.ShapeDtypeStruct((NC, N_PAD, 128), jnp.float32),
        scratch_types=[
            pltpu.VMEM((nch, cb), jnp.int32),
            pltpu.VMEM((2, cb, 128), jnp.float32),
            pltpu.VMEM_SHARED((N_PAD, 128), jnp.float32),
            pltpu.SemaphoreType.DMA,
            pltpu.SemaphoreType.DMA,
        ],
    )
    def k(msg_hbm, dst_hbm, prev_hbm, out_hbm, idx_v, msg_v, acc_sh,
          sl0, sl1):
        cid = lax.axis_index("c")
        sid = lax.axis_index("s")
        wid = sid * NC + cid
        base = wid * epw
        pltpu.sync_copy(prev_hbm.at[cid, pl.ds(sid * NPS, NPS)],
                        acc_sh.at[pl.ds(sid * NPS, NPS)])
        plsc.subcore_barrier()
        pltpu.sync_copy(dst_hbm.at[wid], idx_v)
        pltpu.async_copy(msg_hbm.at[pl.ds(base, cb)], msg_v.at[0], sl0)

        def pair(p, _):
            j0 = 2 * p
            j1 = j0 + 1
            pltpu.async_copy(msg_hbm.at[pl.ds(base + j1 * cb, cb)],
                             msg_v.at[1], sl1)
            pltpu.make_async_copy(msg_hbm.at[pl.ds(base + j0 * cb, cb)],
                                  msg_v.at[0], sl0).wait()
            pltpu.sync_copy(msg_v.at[0], acc_sh.at[idx_v.at[j0]], add=True)

            @pl.when(j1 + 1 < nch)
            def _():
                pltpu.async_copy(msg_hbm.at[pl.ds(base + (j1 + 1) * cb, cb)],
                                 msg_v.at[0], sl0)

            pltpu.make_async_copy(msg_hbm.at[pl.ds(base + j1 * cb, cb)],
                                  msg_v.at[1], sl1).wait()
            pltpu.sync_copy(msg_v.at[1], acc_sh.at[idx_v.at[j1]], add=True)
            return 0

        lax.fori_loop(0, nch // 2, pair, 0)
        if nch % 2:
            j = nch - 1
            pltpu.make_async_copy(msg_hbm.at[pl.ds(base + j * cb, cb)],
                                  msg_v.at[0], sl0).wait()
            pltpu.sync_copy(msg_v.at[0], acc_sh.at[idx_v.at[j]], add=True)
        plsc.subcore_barrier()
        pltpu.sync_copy(acc_sh.at[pl.ds(sid * NPS, NPS)],
                        out_hbm.at[cid, pl.ds(sid * NPS, NPS)])

    return k(msg, dst_r, prev)


def _msg_body(ef_ref, xs_ref, w1_ref, b1_ref, w2_ref, b2_ref, erep_ref,
              esum_ref, out_ref):
    h = jnp.maximum(
        jnp.dot(ef_ref[...], w1_ref[...], preferred_element_type=jnp.float32)
        + b1_ref[...], 0.0)
    we = jnp.dot(h.astype(jnp.bfloat16), w2_ref[...],
                 preferred_element_type=jnp.float32).astype(jnp.bfloat16) + b2_ref[...]
    xr = jnp.dot(xs_ref[...].astype(jnp.bfloat16), erep_ref[...],
                 preferred_element_type=jnp.float32).astype(jnp.bfloat16)
    out_ref[...] = jnp.dot(we * xr, esum_ref[...],
                           preferred_element_type=jnp.float32)


def _tc_msg(edge_feats, xs_h, e0, cnt, be, W1, b1, W2, b2):
    grid = (cnt // be,)
    f = jnp.arange(H)
    erep = (f[None, :] // D_OUT == jnp.arange(128)[:, None]).astype(jnp.bfloat16)
    esum = (f[:, None] % D_OUT == jnp.arange(128)[None, :]).astype(jnp.bfloat16)
    return pl.pallas_call(
        _msg_body,
        grid=grid,
        in_specs=[
            pl.BlockSpec((be, D_EDGE), lambda i, e0=e0, be=be: (i + e0 // be, 0)),
            pl.BlockSpec((be, 128), lambda i: (i, 0)),
            pl.BlockSpec((D_EDGE, H), lambda i: (0, 0)),
            pl.BlockSpec((1, H), lambda i: (0, 0)),
            pl.BlockSpec((H, H), lambda i: (0, 0)),
            pl.BlockSpec((1, H), lambda i: (0, 0)),
            pl.BlockSpec((128, H), lambda i: (0, 0)),
            pl.BlockSpec((H, 128), lambda i: (0, 0)),
        ],
        out_specs=pl.BlockSpec((be, 128), lambda i: (i, 0)),
        out_shape=jax.ShapeDtypeStruct((cnt, 128), jnp.float32),
    )(edge_feats, xs_h, W1.T, b1[None, :], W2.T.astype(jnp.bfloat16),
      b2[None, :].astype(jnp.bfloat16), erep, esum)


def _finish_body(aggp_ref, nf_ref, bconv_ref, wr_ref, wz_ref, wn_ref,
                 br_ref, bz_ref, bni_ref, bnh_ref, wres_ref, bres_ref,
                 gamma_ref, beta_ref, out_ref):
    agg = aggp_ref[0, :N, :D_OUT] + aggp_ref[1, :N, :D_OUT]
    x = jnp.maximum(agg + bconv_ref[...], 0.0)
    r = jax.nn.sigmoid(
        jnp.dot(x, wr_ref[...], preferred_element_type=jnp.float32) + br_ref[...])
    z = jax.nn.sigmoid(
        jnp.dot(x, wz_ref[...], preferred_element_type=jnp.float32) + bz_ref[...])
    n = jnp.tanh(
        jnp.dot(x, wn_ref[...], preferred_element_type=jnp.float32)
        + bni_ref[...] + r * bnh_ref[...])
    hnew = (1.0 - z) * n
    out = hnew + jnp.dot(nf_ref[...], wres_ref[...],
                         preferred_element_type=jnp.float32) + bres_ref[...]
    mean = jnp.mean(out, axis=0, keepdims=True)
    var = jnp.mean((out - mean) ** 2, axis=0, keepdims=True)
    out_ref[...] = (out - mean) * lax.rsqrt(var + 1e-5) * gamma_ref[...] + beta_ref[...]


def _tc_finish(aggp, node_feats, b_conv, Wih, Whh, bih, bhh, Wres,
               bres, gamma, beta):
    O = D_OUT
    args = (
        aggp, node_feats, b_conv[None, :],
        Wih[:O].T, Wih[O:2 * O].T, Wih[2 * O:].T,
        (bih[:O] + bhh[:O])[None, :],
        (bih[O:2 * O] + bhh[O:2 * O])[None, :],
        bih[2 * O:][None, :], bhh[2 * O:][None, :],
        Wres.T, bres[None, :], gamma[None, :], beta[None, :],
    )
    return pl.pallas_call(
        _finish_body,
        out_shape=jax.ShapeDtypeStruct((N, D_OUT), jnp.float32),
    )(*args)


def kernel(node_feats, edge_feats, edge_index, W1, b1, W2, b2, b_conv, Wih,
           Whh, bih, bhh, Wres, bres, gamma, beta):
    src = edge_index[0]
    dst = edge_index[1]
    node_pad = jnp.pad(node_feats, ((0, 0), (0, 128 - D_NODE)))

    offs = [0]
    for c, _, _ in E_SPLIT:
        offs.append(offs[-1] + c)
    xss, msgs = [], []
    for i, (cnt, cb, be) in enumerate(E_SPLIT):
        xss.append(_sc_gather(node_pad, src, offs[i], cnt, cb))
    for i, (cnt, cb, be) in enumerate(E_SPLIT):
        msgs.append(_tc_msg(edge_feats, xss[i], offs[i], cnt, be, W1, b1,
                            W2, b2))
    aggp = jnp.zeros((NC, N_PAD, 128), dtype=jnp.float32)
    for i, (cnt, cb, be) in enumerate(E_SPLIT):
        aggp = _sc_scatter(msgs[i], dst[offs[i]:offs[i + 1]], cnt, cb, aggp)
    return _tc_finish(aggp, node_feats, b_conv, Wih, Whh, bih, bhh,
                      Wres, bres, gamma, beta)
